# baseline (speedup 1.0000x reference)
import functools

import jax
import jax.numpy as jnp
from jax import lax
from jax.experimental import pallas as pl
from jax.experimental.pallas import tpu as pltpu
from jax.experimental.pallas import tpu_sc as plsc

EPS = 1e-06
N_BINS = 256
_L = 16
_NC, _NS = 2, 16
_NW = _NC * _NS
_N_ROWS = 163840
_RPW = _N_ROWS // _NW
_R = 128
_G = _RPW // _R


def _sc_body(x_hbm, out_hbm, xin_v, buf_v, pidx_v, sem0, sem1):
    c = lax.axis_index("c")
    s = lax.axis_index("s")
    wid = s * _NC + c
    base = wid * _RPW
    pltpu.sync_copy(x_hbm.at[pl.ds(base, _RPW)], xin_v)

    iota = lax.iota(jnp.int32, _L)
    ones = jnp.full((_L,), 1.0, jnp.float32)
    zeros = jnp.zeros((_L,), jnp.float32)

    def zrow(r, carry):
        for ch in range(N_BINS // _L):
            buf_v[r, pl.ds(ch * _L, _L)] = zeros
        return carry

    lax.fori_loop(0, 2 * _R, zrow, 0)

    sems = (sem0, sem1)

    def outer(go, carry):
        for b in range(2):
            g = go * 2 + b
            sem = sems[b]

            @pl.when(g >= 2)
            def _wait_and_scrub():
                pltpu.make_async_copy(
                    buf_v.at[pl.ds(b * _R, _R)],
                    out_hbm.at[pl.ds(0, _R)],
                    sem,
                ).wait()

                def scrub(j, inner):
                    rows = b * _R + j * _L + iota
                    cols = pidx_v[pl.ds(b * _R + j * _L, _L)]
                    plsc.store_scatter(buf_v, [rows, cols], zeros)
                    return inner

                lax.fori_loop(0, _R // _L, scrub, 0)

            def setone(j, inner):
                xv = xin_v[pl.ds(g * _R + j * _L, _L)]
                xc = jnp.clip(xv, EPS, 1.0 - EPS)
                idx = jnp.clip((xc * N_BINS).astype(jnp.int32), 0, N_BINS - 1)
                pidx_v[pl.ds(b * _R + j * _L, _L)] = idx
                rows = b * _R + j * _L + iota
                plsc.store_scatter(buf_v, [rows, idx], ones)
                return inner

            lax.fori_loop(0, _R // _L, setone, 0)
            pltpu.async_copy(
                buf_v.at[pl.ds(b * _R, _R)],
                out_hbm.at[pl.ds(base + g * _R, _R)],
                sem,
            )
        return carry

    lax.fori_loop(0, _G // 2, outer, 0)

    for b in range(2):
        pltpu.make_async_copy(
            buf_v.at[pl.ds(b * _R, _R)],
            out_hbm.at[pl.ds(0, _R)],
            sems[b],
        ).wait()


_sc_onehot = functools.partial(
    pl.kernel,
    out_type=jax.ShapeDtypeStruct((_N_ROWS, N_BINS), jnp.float32),
    mesh=plsc.VectorSubcoreMesh(
        core_axis_name="c", subcore_axis_name="s",
        num_cores=_NC, num_subcores=_NS,
    ),
    scratch_types=[
        pltpu.VMEM((_RPW,), jnp.float32),
        pltpu.VMEM((2 * _R, N_BINS), jnp.float32),
        pltpu.VMEM((2 * _R,), jnp.int32),
        pltpu.SemaphoreType.DMA,
        pltpu.SemaphoreType.DMA,
    ],
    compiler_params=pltpu.CompilerParams(
        use_tc_tiling_on_sc=False, needs_layout_passes=False),
)(_sc_body)


def kernel(proprio):
    b, t, f = proprio.shape
    x = proprio.reshape(-1)
    out = _sc_onehot(x)
    tokens = out.reshape(b, t, f, N_BINS)
    mask = jnp.ones((b, t, f), dtype=bool)
    return tokens, mask
